# baseline (speedup 1.0000x reference)
import jax
import jax.numpy as jnp
from jax.experimental import pallas as pl
from jax.experimental.pallas import tpu as pltpu

_HAAR_LL_SCALE = 0.35355339059327373


def _haar_ll_kernel(x_ref, o_ref):
    td, hh, hw = o_ref.shape
    x = x_ref[...]
    d = x.reshape(td, 2, 2 * hh, 2 * hw).sum(axis=1)
    h = d.reshape(td, hh, 2, 2 * hw).sum(axis=2)
    w = h.reshape(td, hh, hw, 2).sum(axis=3)
    o_ref[...] = w * jnp.asarray(_HAAR_LL_SCALE, dtype=o_ref.dtype)


def kernel(x):
    B, C, D, H, W = x.shape
    n = B * C * D
    xf = x.reshape(n, H, W)
    TD = 16
    grid = (n // 2) // TD
    out = pl.pallas_call(
        _haar_ll_kernel,
        grid=(grid,),
        in_specs=[pl.BlockSpec((2 * TD, H, W), lambda i: (i, 0, 0))],
        out_specs=pl.BlockSpec((TD, H // 2, W // 2), lambda i: (i, 0, 0)),
        out_shape=jax.ShapeDtypeStruct((n // 2, H // 2, W // 2), x.dtype),
        compiler_params=pltpu.CompilerParams(
            dimension_semantics=("parallel",),
            vmem_limit_bytes=100 * 1024 * 1024,
        ),
        name="haar3d_ll",
    )(xf)
    out = out.reshape(B, C, D // 2, H // 2, W // 2)
    if C == 1:
        out = out.squeeze(1)
    return out


# speedup vs baseline: 78.3453x; 78.3453x over previous
import jax
import jax.numpy as jnp
from jax import lax
from jax.experimental import pallas as pl
from jax.experimental.pallas import tpu as pltpu

_HAAR_LL_SCALE = 0.35355339059327373


def _haar_ll_kernel(x_ref, o_ref):
    td, hh, hw = o_ref.shape
    h = (
        x_ref[pl.ds(0, td, 2), pl.ds(0, hh, 2), :]
        + x_ref[pl.ds(0, td, 2), pl.ds(1, hh, 2), :]
        + x_ref[pl.ds(1, td, 2), pl.ds(0, hh, 2), :]
        + x_ref[pl.ds(1, td, 2), pl.ds(1, hh, 2), :]
    )
    r = lax.broadcasted_iota(jnp.int32, (2 * hw, hw), 0)
    c = lax.broadcasted_iota(jnp.int32, (2 * hw, hw), 1)
    p = jnp.where(r // 2 == c, _HAAR_LL_SCALE, 0.0).astype(jnp.float32)
    m = jnp.dot(
        h.reshape(td * hh, 2 * hw), p, preferred_element_type=jnp.float32
    )
    o_ref[...] = m.reshape(td, hh, hw).astype(o_ref.dtype)


def kernel(x):
    B, C, D, H, W = x.shape
    n = B * C * D
    xf = x.reshape(n, H, W)
    TD = 16
    grid = (n // 2) // TD
    out = pl.pallas_call(
        _haar_ll_kernel,
        grid=(grid,),
        in_specs=[pl.BlockSpec((2 * TD, H, W), lambda i: (i, 0, 0))],
        out_specs=pl.BlockSpec((TD, H // 2, W // 2), lambda i: (i, 0, 0)),
        out_shape=jax.ShapeDtypeStruct((n // 2, H // 2, W // 2), x.dtype),
        compiler_params=pltpu.CompilerParams(
            dimension_semantics=("parallel",),
            vmem_limit_bytes=100 * 1024 * 1024,
        ),
        name="haar3d_ll",
    )(xf)
    out = out.reshape(B, C, D // 2, H // 2, W // 2)
    if C == 1:
        out = out.squeeze(1)
    return out


# speedup vs baseline: 110.4350x; 1.4096x over previous
import jax
import jax.numpy as jnp
from jax import lax
from jax.experimental import pallas as pl
from jax.experimental.pallas import tpu as pltpu

_HAAR_LL_SCALE = 0.35355339059327373


def _haar_ll_kernel(x_ref, o_ref):
    td, hh, hw = o_ref.shape
    h = (
        x_ref[pl.ds(0, td, 2), pl.ds(0, hh, 2), :]
        + x_ref[pl.ds(0, td, 2), pl.ds(1, hh, 2), :]
        + x_ref[pl.ds(1, td, 2), pl.ds(0, hh, 2), :]
        + x_ref[pl.ds(1, td, 2), pl.ds(1, hh, 2), :]
    )
    r = lax.broadcasted_iota(jnp.int32, (2 * hw, hw), 0)
    c = lax.broadcasted_iota(jnp.int32, (2 * hw, hw), 1)
    p = jnp.where(r // 2 == c, _HAAR_LL_SCALE, 0.0).astype(jnp.float32)
    m = jnp.dot(
        h.reshape(td * hh, 2 * hw), p, preferred_element_type=jnp.float32
    )
    o_ref[...] = m.reshape(td, hh, hw).astype(o_ref.dtype)


def kernel(x):
    B, C, D, H, W = x.shape
    n = B * C * D
    xf = x.reshape(n, H, W)
    TD = 64
    grid = (n // 2) // TD
    out = pl.pallas_call(
        _haar_ll_kernel,
        grid=(grid,),
        in_specs=[pl.BlockSpec((2 * TD, H, W), lambda i: (i, 0, 0))],
        out_specs=pl.BlockSpec((TD, H // 2, W // 2), lambda i: (i, 0, 0)),
        out_shape=jax.ShapeDtypeStruct((n // 2, H // 2, W // 2), x.dtype),
        compiler_params=pltpu.CompilerParams(
            dimension_semantics=("parallel",),
            vmem_limit_bytes=100 * 1024 * 1024,
        ),
        name="haar3d_ll",
    )(xf)
    out = out.reshape(B, C, D // 2, H // 2, W // 2)
    if C == 1:
        out = out.squeeze(1)
    return out
